# baseline (speedup 1.0000x reference)
import functools

import jax
import jax.numpy as jnp
from jax import lax
from jax.experimental import pallas as pl
from jax.experimental.pallas import tpu as pltpu
from jax.experimental.pallas import tpu_sc as plsc

_NC = 2
_NS = 16
_NW = _NC * _NS
_L = 16
_D = 64
_HP = 56
_GP = 8


def _tile_sum(rows_ref, st_ref, st_row, roff, per_row):
    f32 = jnp.float32
    bf16 = jnp.bfloat16
    nw = _D // 2 // _L

    def body(j, accs):
        out = []
        for k in range(nw):
            w = rows_ref[roff + j, pl.ds(k * _L, _L)]
            e, o = plsc.unpack(plsc.bitcast(w, bf16),
                               format=plsc.PackFormat.INTERLEAVED)
            out.append(accs[2 * k] + e)
            out.append(accs[2 * k + 1] + o)
        return tuple(out)

    accs = lax.fori_loop(
        0, per_row, body,
        tuple(jnp.zeros((_L,), f32) for _ in range(_D // _L)))
    for k in range(_D // _L):
        st_ref[st_row, pl.ds(k * _L, _L)] = accs[k]


def _make_sc_kernel(B):
    BPW = B // _NW
    UCH = min(256, BPW)
    NCU = BPW // UCH
    HCR = 4
    NCH = BPW // HCR
    GCR = min(32, BPW)
    NCG = BPW // GCR
    RB = max(UCH, HCR * _HP, GCR * _GP)

    mesh = plsc.VectorSubcoreMesh(
        core_axis_name="c", subcore_axis_name="s",
        num_cores=_NC, num_subcores=_NS)
    f32 = jnp.float32
    i32 = jnp.int32
    _DW = _D // 2
    oute = jax.ShapeDtypeStruct((B, _DW), i32)
    outs = jax.ShapeDtypeStruct((B, _D), f32)

    @functools.partial(
        pl.kernel,
        out_type=(oute, outs, oute, outs),
        mesh=mesh,
        compiler_params=pltpu.CompilerParams(
            use_tc_tiling_on_sc=False, needs_layout_passes=False),
        scratch_types=[
            pltpu.VMEM((RB,), i32),
            pltpu.VMEM((RB,), i32),
            pltpu.VMEM((RB, _DW), i32),
            pltpu.VMEM((RB, _DW), i32),
            pltpu.VMEM((4 * HCR, _D), f32),
            pltpu.VMEM((4 * HCR, _D), f32),
            pltpu.VMEM((GCR, _D), f32),
            pltpu.VMEM((GCR, _D), f32),
            pltpu.SemaphoreType.DMA,
            pltpu.SemaphoreType.DMA,
            pltpu.SemaphoreType.DMA,
            pltpu.SemaphoreType.DMA,
            pltpu.SemaphoreType.DMA,
            pltpu.SemaphoreType.DMA,
            pltpu.SemaphoreType.DMA,
        ],
    )
    def sc_kernel(uidx, iidx, hidx, gidx, utab, itab, gtab,
                  ue, hs, ie, gs,
                  idx_b0, idx_b1, rows_b0, rows_b1,
                  h_st0, h_st1, g_st0, g_st1,
                  sem_a, sem_g0, sem_g1, sem_i0, sem_i1, sem_o0, sem_o1):
        wid = lax.axis_index("s") * _NC + lax.axis_index("c")
        base = wid * BPW

        def vreg_gathers(tab, idx_buf, rows_buf, n, sem, wait):
            def go(s, carry):
                off = pl.multiple_of(s * _L, 8)
                iv = idx_buf[pl.ds(off, _L)]
                cp = pltpu.make_async_copy(
                    tab.at[iv], rows_buf.at[pl.ds(off, _L)], sem)
                if wait:
                    cp.wait()
                else:
                    cp.start()
                return carry
            lax.fori_loop(0, n // _L, go, 0)

        for tab, src, dst in ((utab, uidx, ue), (itab, iidx, ie)):
            for c in range(NCU):
                row0 = base + c * UCH
                pltpu.sync_copy(src.at[pl.ds(row0, UCH)],
                                idx_b0.at[pl.ds(0, UCH)])
                vreg_gathers(tab, idx_b0, rows_b0, UCH, sem_a, False)
                vreg_gathers(tab, idx_b0, rows_b0, UCH, sem_a, True)
                pltpu.sync_copy(rows_b0.at[pl.ds(0, UCH)],
                                dst.at[pl.ds(row0, UCH)])

        def pooled_gather(tab, idx_flat, out, per_row, rpc, nch, st_bufs,
                          sratio):
            ipc = per_row * rpc
            wrows = sratio * rpc
            ibase = base * per_row
            idx_bufs = (idx_b0, idx_b1)
            row_bufs = (rows_b0, rows_b1)
            semg = (sem_g0, sem_g1)
            semi = (sem_i0, sem_i1)
            semo = (sem_o0, sem_o1)

            def idx_src(c):
                return idx_flat.at[pl.ds(ibase + c * ipc, ipc)]

            pltpu.sync_copy(idx_src(0), idx_bufs[0].at[pl.ds(0, ipc)])
            vreg_gathers(tab, idx_bufs[0], row_bufs[0], ipc, semg[0], False)
            pltpu.async_copy(idx_src(1), idx_bufs[1].at[pl.ds(0, ipc)],
                             semi[1])

            unroll = 2 * sratio
            def step(g, carry):
                for u in range(unroll):
                    gb = u % 2
                    ogb = 1 - gb
                    sb = u // sratio
                    c = g * unroll + u

                    @pl.when(c + 1 < nch)
                    def _():
                        pltpu.make_async_copy(
                            idx_src(c + 1), idx_bufs[ogb].at[pl.ds(0, ipc)],
                            semi[ogb]).wait()
                        vreg_gathers(tab, idx_bufs[ogb], row_bufs[ogb],
                                     ipc, semg[ogb], False)

                    vreg_gathers(tab, idx_bufs[gb], row_bufs[gb],
                                 ipc, semg[gb], True)

                    if u % sratio == 0:
                        @pl.when(c >= unroll)
                        def _():
                            pltpu.make_async_copy(
                                st_bufs[sb],
                                out.at[pl.ds(
                                    pl.multiple_of(
                                        base + (c - unroll) * rpc, 8),
                                    wrows)],
                                semo[sb]).wait()

                    rows_full = rows_b0 if gb == 0 else rows_b1
                    srow0 = (u % sratio) * rpc

                    def sum_row(r, carry2):
                        _tile_sum(rows_full, st_bufs[sb], srow0 + r,
                                  r * per_row, per_row)
                        return carry2

                    lax.fori_loop(0, rpc, sum_row, 0)

                    if u % sratio == sratio - 1:
                        pltpu.async_copy(
                            st_bufs[sb],
                            out.at[pl.ds(
                                pl.multiple_of(
                                    base + (c - sratio + 1) * rpc, 8),
                                wrows)],
                            semo[sb])

                    @pl.when(c + 2 < nch)
                    def _():
                        pltpu.async_copy(
                            idx_src(c + 2), idx_bufs[gb].at[pl.ds(0, ipc)],
                            semi[gb])
                return carry

            lax.fori_loop(0, nch // unroll, step, 0)
            pltpu.make_async_copy(
                st_bufs[0],
                out.at[pl.ds(base + (nch - unroll) * rpc, wrows)],
                semo[0]).wait()
            pltpu.make_async_copy(
                st_bufs[1],
                out.at[pl.ds(base + (nch - sratio) * rpc, wrows)],
                semo[1]).wait()

        pooled_gather(itab, hidx, hs, _HP, HCR, NCH, (h_st0, h_st1),
                      sratio=4)
        pooled_gather(gtab, gidx, gs, _GP, GCR, NCG, (g_st0, g_st1),
                      sratio=1)

    return sc_kernel


def _tc_towers(hidx, gidx, ue, hs, ie, gs,
               uW1a, uW1b, ub1, uW2, ub2, iW1a, iW1b, ib1, iW2, ib2,
               blk=2048):
    B, hist = hidx.shape
    gen = gidx.shape[1]
    f32 = jnp.float32

    def body(hidx_ref, gidx_ref, ue_ref, hs_ref, ie_ref, gs_ref,
             uW1a_ref, uW1b_ref, ub1_ref, uW2_ref, ub2_ref,
             iW1a_ref, iW1b_ref, ib1_ref, iW2_ref, ib2_ref, out_ref):
        hcnt = jnp.sum((hidx_ref[...] != 0).astype(f32), axis=1, keepdims=True)
        hmean = hs_ref[...] / (hcnt + 1e-8)
        uh = jnp.dot(ue_ref[...].astype(f32), uW1a_ref[...],
                     preferred_element_type=f32)
        uh += jnp.dot(hmean, uW1b_ref[...], preferred_element_type=f32)
        uh = jnp.maximum(uh + ub1_ref[...], 0.0)
        uv = jnp.dot(uh, uW2_ref[...], preferred_element_type=f32) + ub2_ref[...]

        gcnt = jnp.sum((gidx_ref[...] != 0).astype(f32), axis=1, keepdims=True)
        gmean = gs_ref[...] / (gcnt + 1e-8)
        ih = jnp.dot(ie_ref[...].astype(f32), iW1a_ref[...],
                     preferred_element_type=f32)
        ih += jnp.dot(gmean, iW1b_ref[...], preferred_element_type=f32)
        ih = jnp.maximum(ih + ib1_ref[...], 0.0)
        iv = jnp.dot(ih, iW2_ref[...], preferred_element_type=f32) + ib2_ref[...]

        logits = jnp.sum(uv * iv, axis=1)
        out_ref[...] = 1.0 / (1.0 + jnp.exp(-logits))

    grid = B // blk
    row_spec = lambda w: pl.BlockSpec((blk, w), lambda i: (i, 0))
    full_spec = lambda a: pl.BlockSpec(a.shape, lambda i: (0,) * a.ndim)
    return pl.pallas_call(
        body,
        grid=(grid,),
        in_specs=[
            row_spec(hist), row_spec(gen),
            row_spec(_D), row_spec(_D), row_spec(_D), row_spec(_D),
            full_spec(uW1a), full_spec(uW1b), full_spec(ub1),
            full_spec(uW2), full_spec(ub2),
            full_spec(iW1a), full_spec(iW1b), full_spec(ib1),
            full_spec(iW2), full_spec(ib2),
        ],
        out_specs=pl.BlockSpec((blk,), lambda i: (i,)),
        out_shape=jax.ShapeDtypeStruct((B,), f32),
    )(hidx, gidx, ue, hs, ie, gs,
      uW1a, uW1b, ub1, uW2, ub2, iW1a, iW1b, ib1, iW2, ib2)


def kernel(user_indices, history_indices, item_indices, genre_indices,
           item_table, user_table, genre_table,
           uW1, ub1, uW2, ub2, iW1, ib1, iW2, ib2):
    B = user_indices.shape[0]
    hist = history_indices.shape[1]
    gen = genre_indices.shape[1]
    i32 = jnp.int32

    hflat = jnp.concatenate(
        [history_indices.astype(i32),
         jnp.zeros((B, _HP - hist), i32)], axis=1).reshape(-1)
    gflat = jnp.concatenate(
        [genre_indices.astype(i32),
         jnp.zeros((B, _GP - gen), i32)], axis=1).reshape(-1)

    def to_packed(tab):
        v = tab.shape[0]
        return jax.lax.bitcast_convert_type(
            tab.astype(jnp.bfloat16).reshape(v, _D // 2, 2), i32)

    def from_packed(x):
        return jax.lax.bitcast_convert_type(
            x, jnp.bfloat16).reshape(B, _D)

    ue, hs, ie, gs = _make_sc_kernel(B)(
        user_indices.astype(i32), item_indices.astype(i32), hflat, gflat,
        to_packed(user_table), to_packed(item_table), to_packed(genre_table))
    ue, ie = from_packed(ue), from_packed(ie)

    half = _D // 2
    perm = jnp.concatenate(
        [jnp.arange(0, half, 2), jnp.arange(1, half, 2),
         jnp.arange(half, _D, 2), jnp.arange(half + 1, _D, 2)])

    return _tc_towers(
        history_indices.astype(i32), genre_indices.astype(i32),
        ue, hs, ie, gs,
        uW1[:_D], uW1[_D:][perm], ub1.reshape(1, -1), uW2,
        ub2.reshape(1, -1),
        iW1[:_D], iW1[_D:][perm], ib1.reshape(1, -1), iW2,
        ib2.reshape(1, -1))


# speedup vs baseline: 1.1921x; 1.1921x over previous
import functools

import jax
import jax.numpy as jnp
from jax import lax
from jax.experimental import pallas as pl
from jax.experimental.pallas import tpu as pltpu
from jax.experimental.pallas import tpu_sc as plsc

_NC = 2
_NS = 16
_NW = _NC * _NS
_L = 16
_D = 64
_HP = 56
_GP = 8


def _tile_sum(rows_ref, st_ref, st_row, roff, per_row):
    f32 = jnp.float32

    def body(j, accs):
        return tuple(
            accs[k] + rows_ref[roff + j, pl.ds(k * _L, _L)]
            for k in range(_D // _L))

    accs = lax.fori_loop(
        0, per_row, body,
        tuple(jnp.zeros((_L,), f32) for _ in range(_D // _L)))
    for k in range(_D // _L):
        st_ref[st_row, pl.ds(k * _L, _L)] = accs[k]


def _make_sc_kernel(B):
    BPW = B // _NW
    UCH = min(256, BPW)
    NCU = BPW // UCH
    HCR = 4
    NCH = BPW // HCR
    GCR = min(32, BPW)
    NCG = BPW // GCR
    RB = max(UCH, HCR * _HP, GCR * _GP)

    mesh = plsc.VectorSubcoreMesh(
        core_axis_name="c", subcore_axis_name="s",
        num_cores=_NC, num_subcores=_NS)
    f32 = jnp.float32
    i32 = jnp.int32
    out64 = jax.ShapeDtypeStruct((B, _D), f32)

    @functools.partial(
        pl.kernel,
        out_type=(out64, out64, out64, out64),
        mesh=mesh,
        compiler_params=pltpu.CompilerParams(use_tc_tiling_on_sc=False),
        scratch_types=[
            pltpu.VMEM((RB,), i32),
            pltpu.VMEM((RB,), i32),
            pltpu.VMEM((RB, _D), f32),
            pltpu.VMEM((RB, _D), f32),
            pltpu.VMEM((4 * HCR, _D), f32),
            pltpu.VMEM((4 * HCR, _D), f32),
            pltpu.VMEM((GCR, _D), f32),
            pltpu.VMEM((GCR, _D), f32),
            pltpu.SemaphoreType.DMA,
            pltpu.SemaphoreType.DMA,
            pltpu.SemaphoreType.DMA,
            pltpu.SemaphoreType.DMA,
            pltpu.SemaphoreType.DMA,
            pltpu.SemaphoreType.DMA,
            pltpu.SemaphoreType.DMA,
        ],
    )
    def sc_kernel(uidx, iidx, hidx, gidx, utab, itab, gtab,
                  ue, hs, ie, gs,
                  idx_b0, idx_b1, rows_b0, rows_b1,
                  h_st0, h_st1, g_st0, g_st1,
                  sem_a, sem_g0, sem_g1, sem_i0, sem_i1, sem_o0, sem_o1):
        wid = lax.axis_index("s") * _NC + lax.axis_index("c")
        base = wid * BPW

        def vreg_gathers(tab, idx_buf, rows_buf, n, sem, wait):
            def go(s, carry):
                off = pl.multiple_of(s * _L, 8)
                iv = idx_buf[pl.ds(off, _L)]
                cp = pltpu.make_async_copy(
                    tab.at[iv], rows_buf.at[pl.ds(off, _L)], sem)
                if wait:
                    cp.wait()
                else:
                    cp.start()
                return carry
            lax.fori_loop(0, n // _L, go, 0)

        for tab, src, dst in ((utab, uidx, ue), (itab, iidx, ie)):
            for c in range(NCU):
                row0 = base + c * UCH
                pltpu.sync_copy(src.at[pl.ds(row0, UCH)],
                                idx_b0.at[pl.ds(0, UCH)])
                vreg_gathers(tab, idx_b0, rows_b0, UCH, sem_a, False)
                vreg_gathers(tab, idx_b0, rows_b0, UCH, sem_a, True)
                pltpu.sync_copy(rows_b0.at[pl.ds(0, UCH)],
                                dst.at[pl.ds(row0, UCH)])

        def pooled_gather(tab, idx_flat, out, per_row, rpc, nch, st_bufs,
                          sratio):
            ipc = per_row * rpc
            wrows = sratio * rpc
            ibase = base * per_row
            idx_bufs = (idx_b0, idx_b1)
            row_bufs = (rows_b0, rows_b1)
            semg = (sem_g0, sem_g1)
            semi = (sem_i0, sem_i1)
            semo = (sem_o0, sem_o1)

            def idx_src(c):
                return idx_flat.at[pl.ds(ibase + c * ipc, ipc)]

            pltpu.sync_copy(idx_src(0), idx_bufs[0].at[pl.ds(0, ipc)])
            vreg_gathers(tab, idx_bufs[0], row_bufs[0], ipc, semg[0], False)
            pltpu.async_copy(idx_src(1), idx_bufs[1].at[pl.ds(0, ipc)],
                             semi[1])

            unroll = 2 * sratio
            def step(g, carry):
                for u in range(unroll):
                    gb = u % 2
                    ogb = 1 - gb
                    sb = u // sratio
                    c = g * unroll + u

                    @pl.when(c + 1 < nch)
                    def _():
                        pltpu.make_async_copy(
                            idx_src(c + 1), idx_bufs[ogb].at[pl.ds(0, ipc)],
                            semi[ogb]).wait()
                        vreg_gathers(tab, idx_bufs[ogb], row_bufs[ogb],
                                     ipc, semg[ogb], False)

                    vreg_gathers(tab, idx_bufs[gb], row_bufs[gb],
                                 ipc, semg[gb], True)

                    if u % sratio == 0:
                        @pl.when(c >= unroll)
                        def _():
                            pltpu.make_async_copy(
                                st_bufs[sb],
                                out.at[pl.ds(
                                    pl.multiple_of(
                                        base + (c - unroll) * rpc, 8),
                                    wrows)],
                                semo[sb]).wait()

                    rows_full = rows_b0 if gb == 0 else rows_b1
                    srow0 = (u % sratio) * rpc

                    def sum_row(r, carry2):
                        _tile_sum(rows_full, st_bufs[sb], srow0 + r,
                                  r * per_row, per_row)
                        return carry2

                    lax.fori_loop(0, rpc, sum_row, 0)

                    if u % sratio == sratio - 1:
                        pltpu.async_copy(
                            st_bufs[sb],
                            out.at[pl.ds(
                                pl.multiple_of(
                                    base + (c - sratio + 1) * rpc, 8),
                                wrows)],
                            semo[sb])

                    @pl.when(c + 2 < nch)
                    def _():
                        pltpu.async_copy(
                            idx_src(c + 2), idx_bufs[gb].at[pl.ds(0, ipc)],
                            semi[gb])
                return carry

            lax.fori_loop(0, nch // unroll, step, 0)
            pltpu.make_async_copy(
                st_bufs[0],
                out.at[pl.ds(base + (nch - unroll) * rpc, wrows)],
                semo[0]).wait()
            pltpu.make_async_copy(
                st_bufs[1],
                out.at[pl.ds(base + (nch - sratio) * rpc, wrows)],
                semo[1]).wait()

        pooled_gather(itab, hidx, hs, _HP, HCR, NCH, (h_st0, h_st1),
                      sratio=4)
        pooled_gather(gtab, gidx, gs, _GP, GCR, NCG, (g_st0, g_st1),
                      sratio=1)

    return sc_kernel


def _tc_towers(hidx, gidx, ue, hs, ie, gs,
               uW1a, uW1b, ub1, uW2, ub2, iW1a, iW1b, ib1, iW2, ib2,
               blk=2048):
    B, hist = hidx.shape
    gen = gidx.shape[1]
    f32 = jnp.float32

    def body(hidx_ref, gidx_ref, ue_ref, hs_ref, ie_ref, gs_ref,
             uW1a_ref, uW1b_ref, ub1_ref, uW2_ref, ub2_ref,
             iW1a_ref, iW1b_ref, ib1_ref, iW2_ref, ib2_ref, out_ref):
        hcnt = jnp.sum((hidx_ref[...] != 0).astype(f32), axis=1, keepdims=True)
        hmean = hs_ref[...] / (hcnt + 1e-8)
        uh = jnp.dot(ue_ref[...].astype(f32), uW1a_ref[...],
                     preferred_element_type=f32)
        uh += jnp.dot(hmean, uW1b_ref[...], preferred_element_type=f32)
        uh = jnp.maximum(uh + ub1_ref[...], 0.0)
        uv = jnp.dot(uh, uW2_ref[...], preferred_element_type=f32) + ub2_ref[...]

        gcnt = jnp.sum((gidx_ref[...] != 0).astype(f32), axis=1, keepdims=True)
        gmean = gs_ref[...] / (gcnt + 1e-8)
        ih = jnp.dot(ie_ref[...].astype(f32), iW1a_ref[...],
                     preferred_element_type=f32)
        ih += jnp.dot(gmean, iW1b_ref[...], preferred_element_type=f32)
        ih = jnp.maximum(ih + ib1_ref[...], 0.0)
        iv = jnp.dot(ih, iW2_ref[...], preferred_element_type=f32) + ib2_ref[...]

        logits = jnp.sum(uv * iv, axis=1)
        out_ref[...] = 1.0 / (1.0 + jnp.exp(-logits))

    grid = B // blk
    row_spec = lambda w: pl.BlockSpec((blk, w), lambda i: (i, 0))
    full_spec = lambda a: pl.BlockSpec(a.shape, lambda i: (0,) * a.ndim)
    return pl.pallas_call(
        body,
        grid=(grid,),
        in_specs=[
            row_spec(hist), row_spec(gen),
            row_spec(_D), row_spec(_D), row_spec(_D), row_spec(_D),
            full_spec(uW1a), full_spec(uW1b), full_spec(ub1),
            full_spec(uW2), full_spec(ub2),
            full_spec(iW1a), full_spec(iW1b), full_spec(ib1),
            full_spec(iW2), full_spec(ib2),
        ],
        out_specs=pl.BlockSpec((blk,), lambda i: (i,)),
        out_shape=jax.ShapeDtypeStruct((B,), f32),
    )(hidx, gidx, ue, hs, ie, gs,
      uW1a, uW1b, ub1, uW2, ub2, iW1a, iW1b, ib1, iW2, ib2)


def kernel(user_indices, history_indices, item_indices, genre_indices,
           item_table, user_table, genre_table,
           uW1, ub1, uW2, ub2, iW1, ib1, iW2, ib2):
    B = user_indices.shape[0]
    hist = history_indices.shape[1]
    gen = genre_indices.shape[1]
    i32 = jnp.int32

    hflat = jnp.concatenate(
        [history_indices.astype(i32),
         jnp.zeros((B, _HP - hist), i32)], axis=1).reshape(-1)
    gflat = jnp.concatenate(
        [genre_indices.astype(i32),
         jnp.zeros((B, _GP - gen), i32)], axis=1).reshape(-1)

    ue, hs, ie, gs = _make_sc_kernel(B)(
        user_indices.astype(i32), item_indices.astype(i32), hflat, gflat,
        user_table, item_table, genre_table)

    return _tc_towers(
        history_indices.astype(i32), genre_indices.astype(i32),
        ue, hs, ie, gs,
        uW1[:_D], uW1[_D:], ub1.reshape(1, -1), uW2, ub2.reshape(1, -1),
        iW1[:_D], iW1[_D:], ib1.reshape(1, -1), iW2, ib2.reshape(1, -1))
